# baseline (speedup 1.0000x reference)
import functools

import jax
import jax.numpy as jnp
from jax import lax
from jax.experimental import pallas as pl
from jax.experimental.pallas import tpu as pltpu
from jax.experimental.pallas import tpu_sc as plsc

B = 16384
F = 26
V = 100000
FH = 13
NC = 2
NS = 16
NW = NC * NS
BPW = B // NW
EPH = BPW * FH


def _make_body(f_lo, final):
    def body(table_hbm, x_hbm, *args):
        if final:
            part_hbm, out_hbm, xblk_v, idx_v, rows_v, out_v, sem, gsem = args
        else:
            out_hbm, xblk_v, idx_v, rows_v, out_v, sem, gsem = args
            part_hbm = None
        wid = lax.axis_index("s") * NC + lax.axis_index("c")
        base = wid * BPW

        pltpu.sync_copy(x_hbm.at[:, pl.ds(base, BPW)], xblk_v)

        def build(c, _):
            for fl in range(FH):
                x16 = xblk_v[f_lo + fl, pl.ds(c * 16, 16)]
                idx_v[pl.ds(fl * BPW + c * 16, 16)] = x16 + jnp.int32(fl * V)
            return 0

        lax.fori_loop(0, BPW // 16, build, 0)

        pltpu.async_copy(table_hbm.at[idx_v], rows_v, gsem).wait()

        if final:
            pltpu.sync_copy(part_hbm.at[pl.ds(base, BPW)], out_v)

        def reduce_chunk(c, _):
            if final:
                acc = out_v[pl.ds(c * 16, 16)]
            else:
                acc = rows_v[pl.ds(c * 16, 16)]
            for fl in range(0 if final else 1, FH):
                acc = acc + rows_v[pl.ds(fl * BPW + c * 16, 16)]
            if final:
                acc = 1.0 / (1.0 + jnp.exp(-acc))
            out_v[pl.ds(c * 16, 16)] = acc
            return 0

        lax.fori_loop(0, BPW // 16, reduce_chunk, 0)

        pltpu.sync_copy(out_v, out_hbm.at[pl.ds(base, BPW)])

    return body


def _make_kernel(f_lo, final):
    scratch = [
        pltpu.VMEM((F, BPW), jnp.int32),
        pltpu.VMEM((EPH,), jnp.int32),
        pltpu.VMEM((EPH,), jnp.float32),
        pltpu.VMEM((BPW,), jnp.float32),
        pltpu.SemaphoreType.DMA,
        pltpu.SemaphoreType.DMA,
    ]
    return functools.partial(
        pl.kernel,
        out_type=jax.ShapeDtypeStruct((B,), jnp.float32),
        mesh=plsc.VectorSubcoreMesh(
            core_axis_name="c", subcore_axis_name="s",
            num_cores=NC, num_subcores=NS),
        compiler_params=pltpu.CompilerParams(
            needs_layout_passes=False, use_tc_tiling_on_sc=True),
        scratch_types=scratch,
    )(_make_body(f_lo, final))


_kern_lo = _make_kernel(0, final=False)
_kern_hi = _make_kernel(FH, final=True)


def kernel(X, emb_tables):
    xt = X.T
    t2 = lax.optimization_barrier(jnp.squeeze(emb_tables, 2))
    t_lo = t2[:FH].reshape(FH * V)
    t_hi = t2[FH:].reshape(FH * V)
    partial = _kern_lo(t_lo, xt)
    out = _kern_hi(t_hi, xt, partial)
    return out.reshape(B, 1)


# speedup vs baseline: 1.2016x; 1.2016x over previous
import functools

import jax
import jax.numpy as jnp
from jax import lax
from jax.experimental import pallas as pl
from jax.experimental.pallas import tpu as pltpu
from jax.experimental.pallas import tpu_sc as plsc

B = 16384
F = 26
V = 100000
NC = 2
NS = 16
NW = NC * NS
BPW = B // NW
EPW = BPW * F


FH = 13


def _sc_body(table_hbm, x_hbm, out_hbm, xblk_v, idx_v, rows_v, out_v,
             sem, gsem, gsem2):
    wid = lax.axis_index("s") * NC + lax.axis_index("c")
    base = wid * BPW

    pltpu.sync_copy(x_hbm.at[:, pl.ds(base, BPW)], xblk_v)

    def build(f0):
        def chunk(c, _):
            for fl in range(FH):
                f = f0 + fl
                x16 = xblk_v[f, pl.ds(c * 16, 16)]
                idx_v[pl.ds(f * BPW + c * 16, 16)] = x16 + jnp.int32(f * V)
            return 0
        lax.fori_loop(0, BPW // 16, chunk, 0)

    g1 = pltpu.make_async_copy(
        table_hbm.at[idx_v.at[pl.ds(0, FH * BPW)]],
        rows_v.at[pl.ds(0, FH * BPW)], gsem)
    g2 = pltpu.make_async_copy(
        table_hbm.at[idx_v.at[pl.ds(FH * BPW, FH * BPW)]],
        rows_v.at[pl.ds(FH * BPW, FH * BPW)], gsem2)

    build(0)
    g1.start()
    build(FH)
    g2.start()

    g1.wait()

    def reduce1(c, _):
        acc = rows_v[pl.ds(c * 16, 16)]
        for f in range(1, FH):
            acc = acc + rows_v[pl.ds(f * BPW + c * 16, 16)]
        out_v[pl.ds(c * 16, 16)] = acc
        return 0

    lax.fori_loop(0, BPW // 16, reduce1, 0)

    g2.wait()

    def reduce2(c, _):
        acc = out_v[pl.ds(c * 16, 16)]
        for f in range(FH, F):
            acc = acc + rows_v[pl.ds(f * BPW + c * 16, 16)]
        out_v[pl.ds(c * 16, 16)] = 1.0 / (1.0 + jnp.exp(-acc))
        return 0

    lax.fori_loop(0, BPW // 16, reduce2, 0)

    pltpu.sync_copy(out_v, out_hbm.at[pl.ds(base, BPW)])


_sc_kernel = functools.partial(
    pl.kernel,
    out_type=jax.ShapeDtypeStruct((B,), jnp.float32),
    mesh=plsc.VectorSubcoreMesh(
        core_axis_name="c", subcore_axis_name="s",
        num_cores=NC, num_subcores=NS),
    compiler_params=pltpu.CompilerParams(
        needs_layout_passes=False, use_tc_tiling_on_sc=True),
    scratch_types=[
        pltpu.VMEM((F, BPW), jnp.int32),
        pltpu.VMEM((EPW,), jnp.int32),
        pltpu.VMEM((EPW,), jnp.float32),
        pltpu.VMEM((BPW,), jnp.float32),
        pltpu.SemaphoreType.DMA,
        pltpu.SemaphoreType.DMA,
        pltpu.SemaphoreType.DMA,
    ],
)(_sc_body)


def kernel(X, emb_tables):
    t2 = jnp.squeeze(emb_tables, 2)
    t2 = lax.optimization_barrier(t2)
    table_flat = t2.reshape(F * V)
    out = _sc_kernel(table_flat, X.T)
    return out.reshape(B, 1)
